# baseline (speedup 1.0000x reference)
import functools

import jax
import jax.numpy as jnp
from jax import lax
from jax.experimental import pallas as pl
from jax.experimental.pallas import tpu as pltpu
from jax.experimental.pallas import tpu_sc as plsc

B = 16384
D = 64
NC, NS, L = 2, 16, 16
NW = NC * NS
BPW = B // NW
NCHUNK = 4
CHUNK = BPW // NCHUNK
GROUPS = CHUNK // L

_mesh = plsc.VectorSubcoreMesh(core_axis_name="c", subcore_axis_name="s")


@functools.partial(
    pl.kernel,
    out_type=jax.ShapeDtypeStruct((B,), jnp.float32),
    mesh=_mesh,
    scratch_types=[
        pltpu.VMEM((NCHUNK, CHUNK), jnp.int32),
        pltpu.VMEM((NCHUNK, CHUNK), jnp.int32),
        pltpu.VMEM((NCHUNK, CHUNK, D), jnp.float32),
        pltpu.VMEM((NCHUNK, CHUNK, D), jnp.float32),
        pltpu.VMEM((BPW,), jnp.float32),
        pltpu.SemaphoreType.DMA,
    ],
    compiler_params=pltpu.CompilerParams(
        needs_layout_passes=False, use_tc_tiling_on_sc=False),
)
def _sc_dots(emb_hbm, x1_hbm, x2_hbm, out_hbm, idx1, idx2, rows1, rows2,
             dots, sem):
    wid = lax.axis_index("s") * NC + lax.axis_index("c")
    base = wid * BPW

    for j in range(NCHUNK):
        pltpu.sync_copy(x1_hbm.at[pl.ds(base + j * CHUNK, CHUNK)], idx1.at[j])
        pltpu.sync_copy(x2_hbm.at[pl.ds(base + j * CHUNK, CHUNK)], idx2.at[j])

    copies = []
    for j in range(NCHUNK):
        copies.append(pltpu.async_copy(emb_hbm.at[idx1.at[j]], rows1.at[j], sem))
        copies.append(pltpu.async_copy(emb_hbm.at[idx2.at[j]], rows2.at[j], sem))
    for c in copies:
        c.wait()

    iota = lax.iota(jnp.int32, L)

    for j in range(NCHUNK):
        jvec = jnp.full((L,), j, jnp.int32)

        def group_body(t, _, jvec=jvec, j=j):
            row = t * L + iota
            acc = jnp.zeros((L,), jnp.float32)
            for d in range(D):
                col = jnp.bitwise_and(iota + d, D - 1)
                a = plsc.load_gather(rows1, [jvec, row, col])
                b = plsc.load_gather(rows2, [jvec, row, col])
                acc = acc + a * b
            dots[pl.ds(j * CHUNK + t * L, L)] = acc
            return 0

        lax.fori_loop(0, GROUPS, group_body, 0)

    pltpu.sync_copy(dots, out_hbm.at[pl.ds(base, BPW)])


def _loss_kernel(d_ref, w_ref, o_ref):
    x = w_ref[...] * d_ref[...]
    y = jnp.minimum(x, 0.0) - jnp.log1p(jnp.exp(-jnp.abs(x)))
    o_ref[0, 0] = -jnp.sum(y) * (1.0 / B)


def _loss(dots, w):
    out = pl.pallas_call(
        _loss_kernel,
        out_shape=jax.ShapeDtypeStruct((1, 1), jnp.float32),
        out_specs=pl.BlockSpec(memory_space=pltpu.SMEM),
    )(dots.reshape(128, 128), w.reshape(128, 128))
    return out[0, 0]


def kernel(x1, x2, w, emb):
    dots = _sc_dots(emb, x1.astype(jnp.int32), x2.astype(jnp.int32))
    return _loss(dots, w)


# speedup vs baseline: 1.0023x; 1.0023x over previous
import functools

import jax
import jax.numpy as jnp
from jax import lax
from jax.experimental import pallas as pl
from jax.experimental.pallas import tpu as pltpu
from jax.experimental.pallas import tpu_sc as plsc

B = 16384
D = 64
NC, NS, L = 2, 16, 16
NW = NC * NS
BPW = B // NW
CG = 128
NCHUNK = BPW // CG
GPC = CG // L

_mesh = plsc.VectorSubcoreMesh(core_axis_name="c", subcore_axis_name="s")


@functools.partial(
    pl.kernel,
    out_type=jax.ShapeDtypeStruct((B,), jnp.float32),
    mesh=_mesh,
    scratch_types=[
        pltpu.VMEM((BPW,), jnp.int32),
        pltpu.VMEM((BPW,), jnp.int32),
        pltpu.VMEM((BPW,), jnp.int32),
        pltpu.VMEM((BPW,), jnp.int32),
        pltpu.VMEM((2, CG, 128), jnp.float32),
        pltpu.VMEM((2, CG, 128), jnp.float32),
        pltpu.VMEM((BPW,), jnp.float32),
        pltpu.SemaphoreType.DMA((2,)),
    ],
    compiler_params=pltpu.CompilerParams(needs_layout_passes=False),
)
def _sc_dots(emb2_hbm, x1_hbm, x2_hbm, out_hbm, ix1, ix2, g1, g2,
             rows1, rows2, dots, sems):
    wid = lax.axis_index("s") * NC + lax.axis_index("c")
    base = wid * BPW

    pltpu.sync_copy(x1_hbm.at[pl.ds(base, BPW)], ix1)
    pltpu.sync_copy(x2_hbm.at[pl.ds(base, BPW)], ix2)
    for k in range(BPW // L):
        sl = pl.ds(k * L, L)
        g1[sl] = jnp.right_shift(ix1[sl], 1)
        g2[sl] = jnp.right_shift(ix2[sl], 1)

    def fire(j, slot):
        isl = pl.ds(j * CG, CG)
        pltpu.async_copy(emb2_hbm.at[g1.at[isl]], rows1.at[slot],
                         sems.at[slot])
        pltpu.async_copy(emb2_hbm.at[g2.at[isl]], rows2.at[slot],
                         sems.at[slot])

    def drain(slot):
        dummy = emb2_hbm.at[pl.ds(0, CG)]
        pltpu.make_async_copy(dummy, rows1.at[slot], sems.at[slot]).wait()
        pltpu.make_async_copy(dummy, rows2.at[slot], sems.at[slot]).wait()

    iota = lax.iota(jnp.int32, L)
    fire(0, 0)

    for j in range(NCHUNK):
        slot = j % 2
        drain(slot)
        if j + 1 < NCHUNK:
            fire(j + 1, (j + 1) % 2)

        def group_body(h, _, j=j, slot=slot):
            off = pl.ds(j * CG + h * L, L)
            iv = h * L + iota
            s1 = jnp.left_shift(jnp.bitwise_and(ix1[off], 1), 6)
            s2 = jnp.left_shift(jnp.bitwise_and(ix2[off], 1), 6)
            acc = jnp.zeros((L,), jnp.float32)
            for d in range(D):
                skew = jnp.bitwise_and(iota + d, D - 1)
                a = plsc.load_gather(rows1, [jnp.full((L,), slot, jnp.int32),
                                             iv, s1 + skew])
                b = plsc.load_gather(rows2, [jnp.full((L,), slot, jnp.int32),
                                             iv, s2 + skew])
                acc = acc + a * b
            dots[off] = acc
            return 0

        lax.fori_loop(0, GPC, group_body, 0)

    pltpu.sync_copy(dots, out_hbm.at[pl.ds(base, BPW)])


def _loss_kernel(d_ref, w_ref, o_ref):
    x = w_ref[...] * d_ref[...]
    y = jnp.minimum(x, 0.0) - jnp.log1p(jnp.exp(-jnp.abs(x)))
    o_ref[0, 0] = -jnp.sum(y) * (1.0 / B)


def _loss(dots, w):
    out = pl.pallas_call(
        _loss_kernel,
        out_shape=jax.ShapeDtypeStruct((1, 1), jnp.float32),
        out_specs=pl.BlockSpec(memory_space=pltpu.SMEM),
    )(dots.reshape(128, 128), w.reshape(128, 128))
    return out[0, 0]


def kernel(x1, x2, w, emb):
    emb2 = emb.reshape(500000, 128)
    dots = _sc_dots(emb2, x1.astype(jnp.int32), x2.astype(jnp.int32))
    return _loss(dots, w)
